# baseline (speedup 1.0000x reference)
import functools

import jax
import jax.numpy as jnp
from jax.experimental import pallas as pl
from jax.experimental.pallas import tpu as pltpu

N_TOKENS = 16384
DIM = 4096
N_EXPERTS = 64
TOP_K = 8

BLOCK_T = 512


def _router_body(x_ref, wt_ref, bias_ref, weight_ref, idx_ref):
    x = x_ref[...]
    wt = wt_ref[...]
    score = jax.lax.dot_general(
        x, wt, (((1,), (0,)), ((), ())),
        preferred_element_type=jnp.float32,
    )

    m = jnp.max(score, axis=1, keepdims=True)
    e = jnp.exp(score - m)
    ori = e / jnp.sum(e, axis=1, keepdims=True)

    biased = ori + bias_ref[...]

    iota_f = jax.lax.broadcasted_iota(jnp.int32, biased.shape, 1).astype(jnp.float32)
    neg_inf = jnp.float32(-jnp.inf)
    big = jnp.float32(N_EXPERTS)
    ajs = []
    wjs = []
    for j in range(TOP_K):
        mj = jnp.max(biased, axis=1, keepdims=True)
        cand = jnp.where(biased == mj, iota_f, big)
        aj = jnp.min(cand, axis=1, keepdims=True)
        onehot = iota_f == aj
        wj = jnp.sum(jnp.where(onehot, ori, 0.0), axis=1, keepdims=True)
        ajs.append(aj)
        wjs.append(wj)
        biased = jnp.where(onehot, neg_inf, biased)
    idx_ref[...] = jnp.concatenate(ajs, axis=1).astype(jnp.int32)
    weight_ref[...] = jnp.concatenate(wjs, axis=1)


@jax.jit
def _router(x, wt, bias):
    grid = (N_TOKENS // BLOCK_T,)
    return pl.pallas_call(
        _router_body,
        grid=grid,
        in_specs=[
            pl.BlockSpec((BLOCK_T, DIM), lambda i: (i, 0)),
            pl.BlockSpec((DIM, N_EXPERTS), lambda i: (0, 0)),
            pl.BlockSpec((1, N_EXPERTS), lambda i: (0, 0)),
        ],
        out_specs=[
            pl.BlockSpec((BLOCK_T, TOP_K), lambda i: (i, 0)),
            pl.BlockSpec((BLOCK_T, TOP_K), lambda i: (i, 0)),
        ],
        out_shape=[
            jax.ShapeDtypeStruct((N_TOKENS, TOP_K), jnp.float32),
            jax.ShapeDtypeStruct((N_TOKENS, TOP_K), jnp.int32),
        ],
        compiler_params=pltpu.CompilerParams(
            dimension_semantics=("arbitrary",),
        ),
    )(x, wt, bias)


def kernel(x, W, bias):
    weight, idx = _router(x, W.T, bias.reshape(1, N_EXPERTS))
    return (weight, idx, jnp.float32(0.0))


# speedup vs baseline: 1.0825x; 1.0825x over previous
import functools

import jax
import jax.numpy as jnp
from jax.experimental import pallas as pl
from jax.experimental.pallas import tpu as pltpu

N_TOKENS = 16384
DIM = 4096
N_EXPERTS = 64
TOP_K = 8

BLOCK_T = 1024


def _router_body(x_ref, wt_ref, bias_ref, weight_ref, idx_ref):
    x = x_ref[...]
    wt = wt_ref[...]
    score = jax.lax.dot_general(
        x, wt, (((1,), (0,)), ((), ())),
        preferred_element_type=jnp.float32,
    )

    m = jnp.max(score, axis=1, keepdims=True)
    e = jnp.exp(score - m)
    ori = e / jnp.sum(e, axis=1, keepdims=True)

    biased = ori + bias_ref[...]

    iota_f = jax.lax.broadcasted_iota(jnp.int32, biased.shape, 1).astype(jnp.float32)
    neg_inf = jnp.float32(-jnp.inf)
    big = jnp.float32(N_EXPERTS)
    ajs = []
    wjs = []
    for j in range(TOP_K):
        mj = jnp.max(biased, axis=1, keepdims=True)
        cand = jnp.where(biased == mj, iota_f, big)
        aj = jnp.min(cand, axis=1, keepdims=True)
        onehot = iota_f == aj
        wj = jnp.sum(jnp.where(onehot, ori, 0.0), axis=1, keepdims=True)
        ajs.append(aj)
        wjs.append(wj)
        biased = jnp.where(onehot, neg_inf, biased)
    idx_ref[...] = jnp.concatenate(ajs, axis=1).astype(jnp.int32)
    weight_ref[...] = jnp.concatenate(wjs, axis=1)


@jax.jit
def _router(x, wt, bias):
    grid = (N_TOKENS // BLOCK_T,)
    return pl.pallas_call(
        _router_body,
        grid=grid,
        in_specs=[
            pl.BlockSpec((BLOCK_T, DIM), lambda i: (i, 0)),
            pl.BlockSpec((DIM, N_EXPERTS), lambda i: (0, 0)),
            pl.BlockSpec((1, N_EXPERTS), lambda i: (0, 0)),
        ],
        out_specs=[
            pl.BlockSpec((BLOCK_T, TOP_K), lambda i: (i, 0)),
            pl.BlockSpec((BLOCK_T, TOP_K), lambda i: (i, 0)),
        ],
        out_shape=[
            jax.ShapeDtypeStruct((N_TOKENS, TOP_K), jnp.float32),
            jax.ShapeDtypeStruct((N_TOKENS, TOP_K), jnp.int32),
        ],
        compiler_params=pltpu.CompilerParams(
            dimension_semantics=("arbitrary",),
        ),
    )(x, wt, bias)


def kernel(x, W, bias):
    weight, idx = _router(x, W.T, bias.reshape(1, N_EXPERTS))
    return (weight, idx, jnp.float32(0.0))


# speedup vs baseline: 1.1401x; 1.0532x over previous
import jax
import jax.numpy as jnp
from jax.experimental import pallas as pl
from jax.experimental.pallas import tpu as pltpu

N_TOKENS = 16384
DIM = 4096
N_EXPERTS = 64
TOP_K = 8

BLOCK_T = 1024
CHUNK_T = 256


def _topk_rows(score, bias_row):
    m = jnp.max(score, axis=1, keepdims=True)
    e = jnp.exp(score - m)
    ori = e / jnp.sum(e, axis=1, keepdims=True)
    biased = ori + bias_row

    iota_f = jax.lax.broadcasted_iota(jnp.int32, biased.shape, 1).astype(
        jnp.float32)
    neg_inf = jnp.float32(-jnp.inf)
    big = jnp.float32(N_EXPERTS)
    ajs = []
    wjs = []
    for _ in range(TOP_K):
        mj = jnp.max(biased, axis=1, keepdims=True)
        cand = jnp.where(biased == mj, iota_f, big)
        aj = jnp.min(cand, axis=1, keepdims=True)
        onehot = iota_f == aj
        wj = jnp.sum(jnp.where(onehot, ori, 0.0), axis=1, keepdims=True)
        ajs.append(aj)
        wjs.append(wj)
        biased = jnp.where(onehot, neg_inf, biased)
    weight = jnp.concatenate(wjs, axis=1)
    idx = jnp.concatenate(ajs, axis=1).astype(jnp.int32)
    return weight, idx


def _router_body(x_ref, wt_ref, bias_ref, weight_ref, idx_ref):
    wt = wt_ref[...]
    bias_row = bias_ref[...]
    for r in range(0, BLOCK_T, CHUNK_T):
        x = x_ref[pl.ds(r, CHUNK_T), :]
        score = jax.lax.dot_general(
            x, wt, (((1,), (0,)), ((), ())),
            preferred_element_type=jnp.float32,
        )
        weight, idx = _topk_rows(score, bias_row)
        weight_ref[pl.ds(r, CHUNK_T), :] = weight
        idx_ref[pl.ds(r, CHUNK_T), :] = idx


@jax.jit
def _router(x, wt, bias):
    grid = (N_TOKENS // BLOCK_T,)
    return pl.pallas_call(
        _router_body,
        grid=grid,
        in_specs=[
            pl.BlockSpec((BLOCK_T, DIM), lambda i: (i, 0)),
            pl.BlockSpec((DIM, N_EXPERTS), lambda i: (0, 0)),
            pl.BlockSpec((1, N_EXPERTS), lambda i: (0, 0)),
        ],
        out_specs=[
            pl.BlockSpec((BLOCK_T, TOP_K), lambda i: (i, 0)),
            pl.BlockSpec((BLOCK_T, TOP_K), lambda i: (i, 0)),
        ],
        out_shape=[
            jax.ShapeDtypeStruct((N_TOKENS, TOP_K), jnp.float32),
            jax.ShapeDtypeStruct((N_TOKENS, TOP_K), jnp.int32),
        ],
        compiler_params=pltpu.CompilerParams(
            dimension_semantics=("arbitrary",),
        ),
    )(x, wt, bias)


def kernel(x, W, bias):
    weight, idx = _router(x, W.T, bias.reshape(1, N_EXPERTS))
    return (weight, idx, jnp.float32(0.0))
